# baseline (speedup 1.0000x reference)
import functools

import jax
import jax.numpy as jnp
from jax import lax
from jax.experimental import pallas as pl
from jax.experimental.pallas import tpu as pltpu
from jax.experimental.pallas import tpu_sc as plsc

_D = 64
_C = 128
_NBUF = 4


def _transpose_cols(inb, tb, ncols):
    @plsc.parallel_loop(0, ncols, unroll=16)
    def _(i):
        ivec = jnp.full((16,), i, jnp.int32)
        for k in range(_D // 16):
            dvec = lax.iota(jnp.int32, 16) + 16 * k
            v = plsc.load_gather(inb, [dvec, ivec])
            tb[pl.ds(i * _D + 16 * k, 16)] = v


@functools.lru_cache(maxsize=None)
def _build_kw(V):
    info = plsc.get_sparse_core_info()
    NC, NS = info.num_cores, info.num_subcores
    NW = NC * NS
    NFULL = V // 128
    TAIL = V - NFULL * 128
    n_base = NFULL // NW
    n_extra = NFULL - n_base * NW
    n_max = n_base + (1 if n_extra else 0)
    if n_max % 2:
        n_max += 1
    mesh = plsc.VectorSubcoreMesh(core_axis_name="c", subcore_axis_name="s")

    @functools.partial(
        pl.kernel,
        mesh=mesh,
        out_type=jax.ShapeDtypeStruct((V * _D,), jnp.float32),
        scratch_types=[
            pltpu.VMEM((_D, 128), jnp.float32),
            pltpu.VMEM((_D, 128), jnp.float32),
            pltpu.VMEM((128 * _D,), jnp.float32),
            pltpu.VMEM((128 * _D,), jnp.float32),
        ]
        + [pltpu.SemaphoreType.DMA] * 4,
        compiler_params=pltpu.CompilerParams(
            use_tc_tiling_on_sc=True,
            needs_layout_passes=False,
            disable_bounds_checks=True,
        ),
    )
    def kw(wt_hbm, tail_hbm, out_hbm, inb0, inb1, tb0, tb1, *sems):
        inbs, tbs = (inb0, inb1), (tb0, tb1)
        rsems, wsems = sems[:2], sems[2:]
        wid = lax.axis_index("s") * NC + lax.axis_index("c")
        nblk = n_base + jnp.where(wid < n_extra, 1, 0)
        t0 = wid * n_base + jnp.minimum(wid, n_extra)

        def fire_read(g, sl):
            c0 = pl.multiple_of((t0 + g) * 128, 128)
            pltpu.async_copy(wt_hbm.at[:, pl.ds(c0, 128)], inbs[sl], rsems[sl])

        def wait_write(sl):
            pltpu.make_async_copy(
                tbs[sl], out_hbm.at[pl.ds(0, 128 * _D)], wsems[sl]
            ).wait()

        for k in range(2):

            @pl.when(k < nblk)
            def _():
                fire_read(k, k)

        @pl.loop(0, n_max, step=2)
        def _(g0):
            for sl in range(2):
                g = g0 + sl

                @pl.when(g < nblk)
                def _():
                    pltpu.make_async_copy(
                        wt_hbm.at[:, pl.ds(0, 128)], inbs[sl], rsems[sl]
                    ).wait()

                    @pl.when(g >= 2)
                    def _():
                        wait_write(sl)

                    _transpose_cols(inbs[sl], tbs[sl], 128)
                    c0 = pl.multiple_of((t0 + g) * 128, 128)
                    pltpu.async_copy(
                        tbs[sl], out_hbm.at[pl.ds(c0 * _D, 128 * _D)], wsems[sl]
                    )

                    @pl.when(g + 2 < nblk)
                    def _():
                        fire_read(g + 2, sl)

        for sl in range(2):
            wait_write(sl)

        @pl.when(wid == NW - 1)
        def _():
            pltpu.sync_copy(tail_hbm, tb0.at[pl.ds(0, TAIL * _D)])
            pltpu.sync_copy(
                tb0.at[pl.ds(0, TAIL * _D)],
                out_hbm.at[pl.ds(NFULL * 128 * _D, TAIL * _D)],
            )

    return kw


@functools.lru_cache(maxsize=None)
def _build_gather(B, S):
    info = plsc.get_sparse_core_info()
    NC, NS = info.num_cores, info.num_subcores
    NW = NC * NS
    NBT = B // _C
    T = S * NBT
    per_w = T // NW
    assert T % NW == 0 and per_w % _NBUF == 0
    mesh = plsc.VectorSubcoreMesh(core_axis_name="c", subcore_axis_name="s")

    @functools.partial(
        pl.kernel,
        mesh=mesh,
        out_type=jax.ShapeDtypeStruct((S, _D // 8, NBT, 8, _C), jnp.float32),
        scratch_types=[
            pltpu.VMEM((per_w * _C,), jnp.int32),
            pltpu.VMEM((_NBUF, _C, _D), jnp.float32),
            pltpu.VMEM((2, _D // 8, 8, _C), jnp.float32),
        ]
        + [pltpu.SemaphoreType.DMA] * (_NBUF + 2),
        compiler_params=pltpu.CompilerParams(
            use_tc_tiling_on_sc=False,
            needs_layout_passes=False,
            disable_bounds_checks=True,
        ),
    )
    def grab(idx_hbm, table_hbm, out_hbm, idx_v, rows_v, tout_v, *sems):
        gsems, wsems = sems[:_NBUF], sems[_NBUF:]
        wid = lax.axis_index("s") * NC + lax.axis_index("c")
        t0 = wid * per_w

        bvecs = [lax.iota(jnp.int32, 16) + 16 * c for c in range(8)]

        pltpu.sync_copy(idx_hbm.at[pl.ds(t0 * _C, per_w * _C)], idx_v)

        def fire(g, slot):
            pltpu.async_copy(
                table_hbm.at[idx_v.at[pl.ds(g * _C, _C)]],
                rows_v.at[slot],
                gsems[slot],
            )

        def out_slice(t):
            s = t // NBT
            bt = lax.rem(t, NBT)
            return out_hbm.at[s, :, bt]

        for k in range(_NBUF):
            fire(k, k)

        @pl.loop(0, per_w, step=_NBUF)
        def _(g0):
            for b in range(_NBUF):
                g = g0 + b
                t = t0 + g
                ws = b % 2
                pltpu.make_async_copy(
                    table_hbm.at[idx_v.at[pl.ds(0, _C)]], rows_v.at[b], gsems[b]
                ).wait()

                @pl.when(g >= 2)
                def _():
                    pltpu.make_async_copy(
                        tout_v.at[ws], out_slice(t), wsems[ws]
                    ).wait()

                rows2d = rows_v.at[b]

                @plsc.parallel_loop(0, _D, unroll=16)
                def _(d):
                    dt = d // 8
                    di = lax.rem(d, 8)
                    dvec = jnp.full((16,), d, jnp.int32)
                    for c in range(8):
                        v = plsc.load_gather(rows2d, [bvecs[c], dvec])
                        tout_v[ws, dt, di, pl.ds(16 * c, 16)] = v

                pltpu.async_copy(tout_v.at[ws], out_slice(t), wsems[ws])

                nf = g + _NBUF

                @pl.when(nf < per_w)
                def _():
                    fire(nf, b)

        for ws in range(2):
            pltpu.make_async_copy(
                tout_v.at[ws], out_hbm.at[0, :, 0], wsems[ws]
            ).wait()

    return grab


def kernel(token_ids, weight):
    B, S = token_ids.shape
    V = weight.shape[0]
    idx_flat = token_ids.T.reshape(-1).astype(jnp.int32)
    tail = weight[(V // 128) * 128 :].reshape(-1)
    table = _build_kw(V)(weight.T, tail).reshape(V, _D)
    out5 = _build_gather(B, S)(idx_flat, table)
    return out5.transpose(2, 4, 0, 1, 3).reshape(B, S, _D)


# speedup vs baseline: 1.4563x; 1.4563x over previous
import functools

import jax
import jax.numpy as jnp
from jax import lax
from jax.experimental import pallas as pl
from jax.experimental.pallas import tpu as pltpu
from jax.experimental.pallas import tpu_sc as plsc

_D = 64
_C = 128
_NBUF = 4


def _transpose_dmaj(inb, tb):
    it = [lax.iota(jnp.int32, 16) + 16 * k for k in range(8)]
    rvecs = [v // 2 for v in it]
    cbase = [lax.rem(v, 2) * _D for v in it]

    @plsc.parallel_loop(0, _D, unroll=16)
    def _(d):
        for k in range(8):
            v = inb[d, pl.ds(16 * k, 16)]
            plsc.store_scatter(tb, [rvecs[k], cbase[k] + d], v)


@functools.lru_cache(maxsize=None)
def _build_kw(V):
    info = plsc.get_sparse_core_info()
    NC, NS = info.num_cores, info.num_subcores
    NW = NC * NS
    NFULL = V // 128
    TAIL = V - NFULL * 128
    n_base = NFULL // NW
    n_extra = NFULL - n_base * NW
    n_max = n_base + (1 if n_extra else 0)
    if n_max % 2:
        n_max += 1
    mesh = plsc.VectorSubcoreMesh(core_axis_name="c", subcore_axis_name="s")

    @functools.partial(
        pl.kernel,
        mesh=mesh,
        out_type=jax.ShapeDtypeStruct((V // 2, 2 * _D), jnp.float32),
        scratch_types=[
            pltpu.VMEM((_D, 128), jnp.float32),
            pltpu.VMEM((_D, 128), jnp.float32),
            pltpu.VMEM((_D, 131), jnp.float32),
            pltpu.VMEM((_D, 131), jnp.float32),
        ]
        + [pltpu.SemaphoreType.DMA] * 4,
        compiler_params=pltpu.CompilerParams(
            use_tc_tiling_on_sc=True,
            needs_layout_passes=False,
            disable_bounds_checks=True,
        ),
    )
    def kw(wt_hbm, tail_hbm, out_hbm, inb0, inb1, tb0, tb1, *sems):
        inbs, tbs = (inb0, inb1), (tb0, tb1)
        rsems, wsems = sems[:2], sems[2:]
        wid = lax.axis_index("s") * NC + lax.axis_index("c")
        nblk = n_base + jnp.where(wid < n_extra, 1, 0)
        t0 = wid * n_base + jnp.minimum(wid, n_extra)

        def fire_read(g, sl):
            c0 = pl.multiple_of((t0 + g) * 128, 128)
            pltpu.async_copy(wt_hbm.at[:, pl.ds(c0, 128)], inbs[sl], rsems[sl])

        def wait_write(sl):
            pltpu.make_async_copy(
                tbs[sl].at[:, pl.ds(0, 2 * _D)],
                out_hbm.at[pl.ds(0, _D)],
                wsems[sl],
            ).wait()

        for k in range(2):

            @pl.when(k < nblk)
            def _():
                fire_read(k, k)

        @pl.loop(0, n_max, step=2)
        def _(g0):
            for sl in range(2):
                g = g0 + sl

                @pl.when(g < nblk)
                def _():
                    pltpu.make_async_copy(
                        wt_hbm.at[:, pl.ds(0, 128)], inbs[sl], rsems[sl]
                    ).wait()

                    @pl.when(g >= 2)
                    def _():
                        wait_write(sl)

                    _transpose_dmaj(inbs[sl], tbs[sl])
                    r0 = pl.multiple_of((t0 + g) * _D, _D)
                    pltpu.async_copy(
                        tbs[sl].at[:, pl.ds(0, 2 * _D)],
                        out_hbm.at[pl.ds(r0, _D)],
                        wsems[sl],
                    )

                    @pl.when(g + 2 < nblk)
                    def _():
                        fire_read(g + 2, sl)

        for sl in range(2):
            wait_write(sl)

        @pl.when(wid == NW - 1)
        def _():
            pltpu.sync_copy(tail_hbm, inb0.at[pl.ds(0, TAIL // 2)])
            pltpu.sync_copy(
                inb0.at[pl.ds(0, TAIL // 2)],
                out_hbm.at[pl.ds(NFULL * _D, TAIL // 2)],
            )

    return kw


@functools.lru_cache(maxsize=None)
def _build_gather(B, S):
    info = plsc.get_sparse_core_info()
    NC, NS = info.num_cores, info.num_subcores
    NW = NC * NS
    NBT = B // _C
    T = S * NBT
    per_w = T // NW
    assert T % NW == 0 and per_w % _NBUF == 0
    mesh = plsc.VectorSubcoreMesh(core_axis_name="c", subcore_axis_name="s")

    @functools.partial(
        pl.kernel,
        mesh=mesh,
        out_type=jax.ShapeDtypeStruct((S, _D // 8, NBT, 8, _C), jnp.float32),
        scratch_types=[
            pltpu.VMEM((per_w * _C,), jnp.int32),
            pltpu.VMEM((_NBUF, _C, _D), jnp.float32),
            pltpu.VMEM((2, _D // 8, 8, _C + 1), jnp.float32),
        ]
        + [pltpu.SemaphoreType.DMA] * (_NBUF + 2),
        compiler_params=pltpu.CompilerParams(
            use_tc_tiling_on_sc=False,
            needs_layout_passes=False,
            disable_bounds_checks=True,
        ),
    )
    def grab(idx_hbm, table_hbm, out_hbm, idx_v, rows_v, tout_v, *sems):
        gsems, wsems = sems[:_NBUF], sems[_NBUF:]
        wid = lax.axis_index("s") * NC + lax.axis_index("c")
        t0 = wid * per_w

        dts = [(lax.iota(jnp.int32, 16) + 16 * k) // 8 for k in range(_D // 16)]
        dis = [lax.rem(lax.iota(jnp.int32, 16) + 16 * k, 8) for k in range(_D // 16)]

        pltpu.sync_copy(idx_hbm.at[pl.ds(t0 * _C, per_w * _C)], idx_v)

        def fire(g, slot):
            pltpu.async_copy(
                table_hbm.at[idx_v.at[pl.ds(g * _C, _C)]],
                rows_v.at[slot],
                gsems[slot],
            )

        def out_slice(t):
            s = t // NBT
            bt = lax.rem(t, NBT)
            return out_hbm.at[s, :, bt]

        for k in range(_NBUF):
            fire(k, k)

        @pl.loop(0, per_w, step=_NBUF)
        def _(g0):
            for b in range(_NBUF):
                g = g0 + b
                t = t0 + g
                ws = b % 2
                pltpu.make_async_copy(
                    table_hbm.at[idx_v.at[pl.ds(0, _C)]], rows_v.at[b], gsems[b]
                ).wait()

                @pl.when(g >= 2)
                def _():
                    pltpu.make_async_copy(
                        tout_v.at[ws].at[:, :, pl.ds(0, _C)],
                        out_slice(t),
                        wsems[ws],
                    ).wait()

                rows2d = rows_v.at[b]
                toutws = tout_v.at[ws]

                @plsc.parallel_loop(0, _C, unroll=16)
                def _(tok):
                    bvec = jnp.full((16,), tok, jnp.int32)
                    for k in range(_D // 16):
                        v = rows2d[tok, pl.ds(16 * k, 16)]
                        plsc.store_scatter(toutws, [dts[k], dis[k], bvec], v)

                pltpu.async_copy(
                    tout_v.at[ws].at[:, :, pl.ds(0, _C)], out_slice(t), wsems[ws]
                )

                nf = g + _NBUF

                @pl.when(nf < per_w)
                def _():
                    fire(nf, b)

        for ws in range(2):
            pltpu.make_async_copy(
                tout_v.at[ws].at[:, :, pl.ds(0, _C)],
                out_hbm.at[0, :, 0],
                wsems[ws],
            ).wait()

    return grab


def kernel(token_ids, weight):
    B, S = token_ids.shape
    V = weight.shape[0]
    idx_flat = token_ids.T.reshape(-1).astype(jnp.int32)
    tail = weight[(V // 128) * 128 :].reshape(-1, 2 * _D)
    table = _build_kw(V)(weight.T, tail).reshape(V, _D)
    out5 = _build_gather(B, S)(idx_flat, table)
    return out5.transpose(2, 4, 0, 1, 3).reshape(B, S, _D)
